# baseline (speedup 1.0000x reference)
import functools

import jax
import jax.numpy as jnp
from jax import lax
from jax.experimental import pallas as pl
from jax.experimental.pallas import tpu as pltpu
from jax.experimental.pallas import tpu_sc as plsc

_NC = 2
_NS = 16
_NW = _NC * _NS
_L = 16

_RW = 4
_IDX_PER_DMA = 128


def _make_kernel(BR, S, V, NP, D, DP):
    rows_w = BR // _NW
    G = rows_w // _RW
    n_grp = (S + _L - 1) // _L
    assert BR % _NW == 0 and rows_w % _RW == 0 and G >= 4
    mesh = plsc.VectorSubcoreMesh(core_axis_name="c", subcore_axis_name="s")

    @functools.partial(
        pl.kernel,
        out_type=jax.ShapeDtypeStruct((BR * S, DP), jnp.float32),
        mesh=mesh,
        compiler_params=pltpu.CompilerParams(
            needs_layout_passes=False, use_tc_tiling_on_sc=False
        ),
        scratch_types=[
            [pltpu.VMEM((_RW, S), jnp.int32) for _ in range(4)],
            [pltpu.VMEM((_RW, S), jnp.int32) for _ in range(2)],
            [pltpu.VMEM((_RW * S, D), jnp.float32) for _ in range(2)],
            pltpu.VMEM((NP, D), jnp.float32),
            [pltpu.SemaphoreType.DMA for _ in range(4)],
            [pltpu.SemaphoreType.DMA for _ in range(2)],
            [pltpu.SemaphoreType.DMA for _ in range(2)],
        ],
    )
    def k(idx_hbm, embed_hbm, ne_hbm, out_hbm, Q, Sv, R, ne_v, si, sg, so):
        wid = lax.axis_index("s") * _NC + lax.axis_index("c")
        base = wid * rows_w
        pltpu.sync_copy(ne_hbm, ne_v)

        def fire_idx(g, q):
            pltpu.async_copy(idx_hbm.at[pl.ds(base + g * _RW, _RW)], Q[q], si[q])

        def wait_idx(q):
            pltpu.make_async_copy(idx_hbm.at[pl.ds(0, _RW)], Q[q], si[q]).wait()

        def remap(q, b):
            for rr in range(_RW):
                def step(j, _):
                    off = jnp.minimum(j * _L, S - _L)
                    v = Q[q][rr, pl.ds(off, _L)]
                    Sv[b][rr, pl.ds(off, _L)] = jnp.where(v >= V, 0, v)
                    return 0

                lax.fori_loop(0, n_grp, step, 0, unroll=4)

        def fire_gathers(b):
            for rr in range(_RW):
                for off in range(0, S, _IDX_PER_DMA):
                    n = min(_IDX_PER_DMA, S - off)
                    pltpu.async_copy(
                        embed_hbm.at[Sv[b].at[rr].at[pl.ds(off, n)]],
                        R[b].at[pl.ds(rr * S + off, n)],
                        sg[b],
                    )

        def wait_gathers(b):
            pltpu.make_async_copy(embed_hbm.at[pl.ds(0, _RW * S)], R[b], sg[b]).wait()

        def fixup(q, b):
            for rr in range(_RW):
                def step(j, _):
                    off = jnp.minimum(j * _L, S - _L)
                    v = Q[q][rr, pl.ds(off, _L)]
                    m = v >= V
                    p = jnp.where(m, v - V, 0)

                    @pl.when(jnp.max(v) >= V)
                    def _():
                        rvec = rr * S + off + lax.iota(jnp.int32, _L)

                        def col(c, _):
                            cc = jnp.full((_L,), c, jnp.int32)
                            vals = plsc.load_gather(ne_v, [p, cc], mask=m)
                            plsc.store_scatter(R[b], [rvec, cc], vals, mask=m)
                            return 0

                        lax.fori_loop(0, D, col, 0)

                    return 0

                lax.fori_loop(0, n_grp, step, 0)

        def fire_store(g, b):
            pltpu.async_copy(
                R[b],
                out_hbm.at[pl.ds((base + g * _RW) * S, _RW * S)].at[:, pl.ds(0, D)],
                so[b],
            )

        def wait_store(b):
            pltpu.make_async_copy(
                R[b], out_hbm.at[pl.ds(0, _RW * S)].at[:, pl.ds(0, D)], so[b]
            ).wait()

        def body(g, q, b, prefetch):
            p, qp = 1 - b, (q - 1) % 4
            wait_idx(q)
            remap(q, b)
            wait_store(b)
            fire_gathers(b)
            wait_gathers(p)
            fixup(qp, p)
            fire_store(g - 1, p)
            if prefetch:
                fire_idx(g + 2, (q + 2) % 4)

        fire_idx(0, 0)
        fire_idx(1, 1)
        wait_idx(0)
        remap(0, 0)
        fire_gathers(0)
        fire_idx(2, 2)
        wait_idx(1)
        remap(1, 1)
        fire_gathers(1)
        wait_gathers(0)
        fixup(0, 0)
        fire_store(0, 0)
        fire_idx(3, 3)

        n_quads = (G - 2) // 4

        def quad(kk, _):
            for j in range(4):
                body(2 + kk * 4 + j, (2 + j) % 4, j % 2, True)
            return 0

        lax.fori_loop(0, n_quads, quad, 0)

        for g in range(2 + n_quads * 4, G):
            body(g, g % 4, g % 2, g + 2 <= G - 1)

        bl = (G - 1) % 2
        wait_gathers(bl)
        fixup((G - 1) % 4, bl)
        fire_store(G - 1, bl)
        wait_store(1 - bl)
        wait_store(bl)

    return k


@jax.jit
def kernel(input, embed_weight, new_embed_weight):
    BR, S = input.shape
    V, D = embed_weight.shape
    NP = new_embed_weight.shape[0]
    DP = 2 * D
    k = _make_kernel(BR, S, V, NP, D, DP)
    out = k(input, embed_weight, new_embed_weight)
    return out[:, :D].reshape(BR, S, D)


# speedup vs baseline: 1.0048x; 1.0048x over previous
import functools

import jax
import jax.numpy as jnp
from jax import lax
from jax.experimental import pallas as pl
from jax.experimental.pallas import tpu as pltpu
from jax.experimental.pallas import tpu_sc as plsc

_NC = 2
_NS = 16
_NW = _NC * _NS
_L = 16

_RW = 2
_IDX_PER_DMA = 128


def _make_kernel(BR, S, V, NP, D, DP):
    rows_w = BR // _NW
    G = rows_w // _RW
    n_grp = (S + _L - 1) // _L
    assert BR % _NW == 0 and rows_w % _RW == 0 and G >= 4
    mesh = plsc.VectorSubcoreMesh(core_axis_name="c", subcore_axis_name="s")

    @functools.partial(
        pl.kernel,
        out_type=jax.ShapeDtypeStruct((BR * S, DP), jnp.float32),
        mesh=mesh,
        compiler_params=pltpu.CompilerParams(
            needs_layout_passes=False, use_tc_tiling_on_sc=False
        ),
        scratch_types=[
            [pltpu.VMEM((_RW, S), jnp.int32) for _ in range(4)],
            [pltpu.VMEM((_RW, S), jnp.int32) for _ in range(2)],
            [pltpu.VMEM((_RW * S, D), jnp.float32) for _ in range(2)],
            pltpu.VMEM((NP, D), jnp.float32),
            [pltpu.SemaphoreType.DMA for _ in range(4)],
            [pltpu.SemaphoreType.DMA for _ in range(2)],
            [pltpu.SemaphoreType.DMA for _ in range(2)],
        ],
    )
    def k(idx_hbm, embed_hbm, ne_hbm, out_hbm, Q, Sv, R, ne_v, si, sg, so):
        wid = lax.axis_index("s") * _NC + lax.axis_index("c")
        base = wid * rows_w
        pltpu.sync_copy(ne_hbm, ne_v)

        def fire_idx(g, q):
            pltpu.async_copy(idx_hbm.at[pl.ds(base + g * _RW, _RW)], Q[q], si[q])

        def wait_idx(q):
            pltpu.make_async_copy(idx_hbm.at[pl.ds(0, _RW)], Q[q], si[q]).wait()

        def remap(q, b):
            for rr in range(_RW):
                def step(j, _):
                    off = jnp.minimum(j * _L, S - _L)
                    v = Q[q][rr, pl.ds(off, _L)]
                    Sv[b][rr, pl.ds(off, _L)] = jnp.where(v >= V, 0, v)
                    return 0

                lax.fori_loop(0, n_grp, step, 0, unroll=4)

        def fire_gathers(b):
            for rr in range(_RW):
                for off in range(0, S, _IDX_PER_DMA):
                    n = min(_IDX_PER_DMA, S - off)
                    pltpu.async_copy(
                        embed_hbm.at[Sv[b].at[rr].at[pl.ds(off, n)]],
                        R[b].at[pl.ds(rr * S + off, n)],
                        sg[b],
                    )

        def wait_gathers(b):
            pltpu.make_async_copy(embed_hbm.at[pl.ds(0, _RW * S)], R[b], sg[b]).wait()

        def fixup(q, b):
            for rr in range(_RW):
                def step(j, _):
                    off = jnp.minimum(j * _L, S - _L)
                    v = Q[q][rr, pl.ds(off, _L)]
                    m = v >= V
                    p = jnp.where(m, v - V, 0)

                    @pl.when(jnp.max(v) >= V)
                    def _():
                        rvec = rr * S + off + lax.iota(jnp.int32, _L)

                        def col(c, _):
                            cc = jnp.full((_L,), c, jnp.int32)
                            vals = plsc.load_gather(ne_v, [p, cc], mask=m)
                            plsc.store_scatter(R[b], [rvec, cc], vals, mask=m)
                            return 0

                        lax.fori_loop(0, D, col, 0)

                    return 0

                lax.fori_loop(0, n_grp, step, 0)

        def fire_store(g, b):
            pltpu.async_copy(
                R[b],
                out_hbm.at[pl.ds((base + g * _RW) * S, _RW * S)].at[:, pl.ds(0, D)],
                so[b],
            )

        def wait_store(b):
            pltpu.make_async_copy(
                R[b], out_hbm.at[pl.ds(0, _RW * S)].at[:, pl.ds(0, D)], so[b]
            ).wait()

        def body(g, q, b, prefetch):
            p, qp = 1 - b, (q - 1) % 4
            wait_idx(q)
            remap(q, b)
            wait_store(b)
            fire_gathers(b)
            wait_gathers(p)
            fixup(qp, p)
            fire_store(g - 1, p)
            if prefetch:
                fire_idx(g + 2, (q + 2) % 4)

        fire_idx(0, 0)
        fire_idx(1, 1)
        wait_idx(0)
        remap(0, 0)
        fire_gathers(0)
        fire_idx(2, 2)
        wait_idx(1)
        remap(1, 1)
        fire_gathers(1)
        wait_gathers(0)
        fixup(0, 0)
        fire_store(0, 0)
        fire_idx(3, 3)

        n_quads = (G - 2) // 4

        def quad(kk, _):
            for j in range(4):
                body(2 + kk * 4 + j, (2 + j) % 4, j % 2, True)
            return 0

        lax.fori_loop(0, n_quads, quad, 0)

        for g in range(2 + n_quads * 4, G):
            body(g, g % 4, g % 2, g + 2 <= G - 1)

        bl = (G - 1) % 2
        wait_gathers(bl)
        fixup((G - 1) % 4, bl)
        fire_store(G - 1, bl)
        wait_store(1 - bl)
        wait_store(bl)

    return k


@jax.jit
def kernel(input, embed_weight, new_embed_weight):
    BR, S = input.shape
    V, D = embed_weight.shape
    NP = new_embed_weight.shape[0]
    DP = 2 * D
    k = _make_kernel(BR, S, V, NP, D, DP)
    out = k(input, embed_weight, new_embed_weight)
    return out[:, :D].reshape(BR, S, D)
